# Initial kernel scaffold; baseline (speedup 1.0000x reference)
import jax
import jax.numpy as jnp
from jax.experimental import pallas as pl


def kernel(features, src0, dst0, src1, dst1, W0, b0, W1, b1):
    raise NotImplementedError("write your pallas kernel here")


# speedup vs baseline: 4.5401x; 4.5401x over previous
import functools

import jax
import jax.numpy as jnp
from jax import lax
from jax.experimental import pallas as pl
from jax.experimental.pallas import tpu as pltpu
from jax.experimental.pallas import tpu_sc as plsc

_N0, _N1, _N2 = 50000, 10000, 1000
_E0, _E1 = 160000, 16000
_D_IN, _D_HID, _D_OUT = 256, 128, 64

_NC, _NS = 2, 16
_BATCH = 128
_CW = 16


def _make_seg_sum(e_pad, dh, s_pad):
    nb = e_pad // (_NS * _BATCH)
    zr = s_pad // _NS
    zc = min(_BATCH, zr)
    mesh = plsc.VectorSubcoreMesh(core_axis_name="c", subcore_axis_name="s")

    @functools.partial(
        pl.kernel,
        mesh=mesh,
        compiler_params=pltpu.CompilerParams(use_tc_tiling_on_sc=False),
        out_type=[
            jax.ShapeDtypeStruct((_NC, s_pad, dh), jnp.float32),
            jax.ShapeDtypeStruct((s_pad, _CW), jnp.float32),
        ],
        scratch_types=[
            pltpu.VMEM((nb, _BATCH), jnp.int32),
            pltpu.VMEM((nb, _BATCH), jnp.int32),
            pltpu.VMEM((_BATCH, dh), jnp.float32),
            pltpu.VMEM((_BATCH, _CW), jnp.float32),
            pltpu.VMEM((_BATCH, _CW), jnp.float32),
            pltpu.SemaphoreType.DMA,
            pltpu.VMEM_SHARED((s_pad, dh), jnp.float32),
            pltpu.VMEM_SHARED((s_pad, _CW), jnp.float32),
        ],
    )
    def seg_kernel(table, src, dst, zeros_d, zeros_c, ones_c, acc_out, cnt_out,
                   src_v, dst_v, rows_v, ones_v, zc_v, sem, acc_sh, cnt_sh):
        c = lax.axis_index("c")
        s = lax.axis_index("s")
        pltpu.sync_copy(zeros_d, rows_v)
        pltpu.sync_copy(zeros_c, zc_v)
        pltpu.sync_copy(ones_c, ones_v)
        for t in range(zr // zc):
            r0 = s * zr + t * zc
            pltpu.sync_copy(rows_v.at[pl.ds(0, zc)], acc_sh.at[pl.ds(r0, zc)])
            pltpu.sync_copy(zc_v.at[pl.ds(0, zc)], cnt_sh.at[pl.ds(r0, zc)])
        plsc.subcore_barrier()
        base = s * nb
        pltpu.sync_copy(src.at[pl.ds(base, nb)], src_v)
        pltpu.sync_copy(dst.at[pl.ds(base, nb)], dst_v)
        tbl = table.at[c]

        def body(j, carry):
            pltpu.async_copy(tbl.at[src_v.at[j]], rows_v, sem).wait()
            pltpu.sync_copy(rows_v, acc_sh.at[dst_v.at[j]], add=True)
            return carry

        lax.fori_loop(0, nb, body, 0)

        @pl.when(c == 0)
        def _():
            def cbody(j, carry):
                pltpu.sync_copy(ones_v, cnt_sh.at[dst_v.at[j]], add=True)
                return carry
            lax.fori_loop(0, nb, cbody, 0)

        plsc.subcore_barrier()
        r0 = s * zr
        pltpu.sync_copy(acc_sh.at[pl.ds(r0, zr)], acc_out.at[c, pl.ds(r0, zr)])

        @pl.when(c == 0)
        def _():
            pltpu.sync_copy(cnt_sh.at[pl.ds(r0, zr)], cnt_out.at[pl.ds(r0, zr)])

    return seg_kernel


def _matmul_split(x, w):
    n, k = x.shape
    h = w.shape[1] // 2
    bm = 2000

    def mk(x_ref, w_ref, o_ref):
        res = jnp.dot(x_ref[...], w_ref[...],
                      preferred_element_type=jnp.float32)
        o_ref[...] = jnp.stack([res[:, :h], res[:, h:]], axis=0)

    return pl.pallas_call(
        mk,
        grid=(n // bm,),
        in_specs=[pl.BlockSpec((bm, k), lambda i: (i, 0)),
                  pl.BlockSpec((k, 2 * h), lambda i: (0, 0))],
        out_specs=pl.BlockSpec((2, bm, h), lambda i: (0, i, 0)),
        out_shape=jax.ShapeDtypeStruct((2, n, h), jnp.float32),
    )(x, w)


def _mid(acc, cnt, b0, w1a, w1b):
    s_pad = acc.shape[1]
    bm = 1024
    h = _D_OUT // 2

    def mk(a_ref, c_ref, b0r, wa, wb, o):
        inv = 1.0 / jnp.maximum(c_ref[...][:, 0:1], 1.0)
        a = (jnp.concatenate([a_ref[0], a_ref[1]], axis=1) * inv + b0r[...])
        res = (jnp.dot(a, wa[...], preferred_element_type=jnp.float32)
               + jnp.dot(jnp.maximum(a, 0.0), wb[...],
                         preferred_element_type=jnp.float32))
        o[...] = jnp.stack([res[:, :h], res[:, h:]], axis=0)

    return pl.pallas_call(
        mk,
        grid=(s_pad // bm,),
        in_specs=[pl.BlockSpec((2, bm, _D_HID // 2), lambda i: (0, i, 0)),
                  pl.BlockSpec((bm, _CW), lambda i: (i, 0)),
                  pl.BlockSpec((1, _D_HID), lambda i: (0, 0)),
                  pl.BlockSpec((_D_HID, _D_OUT), lambda i: (0, 0)),
                  pl.BlockSpec((_D_HID, _D_OUT), lambda i: (0, 0))],
        out_specs=pl.BlockSpec((2, bm, h), lambda i: (0, i, 0)),
        out_shape=jax.ShapeDtypeStruct((2, s_pad, h), jnp.float32),
    )(acc, cnt, b0, w1a, w1b)


def _fin(acc, cnt, b1):
    s_pad = acc.shape[1]

    def mk(a_ref, c_ref, b1r, o):
        inv = 1.0 / jnp.maximum(c_ref[...][:, 0:1], 1.0)
        res = jnp.concatenate([a_ref[0], a_ref[1]], axis=1) * inv + b1r[...]
        o[...] = res[:_N2]

    return pl.pallas_call(
        mk,
        grid=(1,),
        in_specs=[pl.BlockSpec((2, s_pad, _D_OUT // 2), lambda i: (0, 0, 0)),
                  pl.BlockSpec((s_pad, _CW), lambda i: (0, 0)),
                  pl.BlockSpec((1, _D_OUT), lambda i: (0, 0))],
        out_specs=pl.BlockSpec((_N2, _D_OUT), lambda i: (0, 0)),
        out_shape=jax.ShapeDtypeStruct((_N2, _D_OUT), jnp.float32),
    )(acc, cnt, b1)


_E0_PAD = 163840
_E1_PAD = 16384
_S0_PAD = 10240
_S1_PAD = 1024


@functools.lru_cache(maxsize=None)
def _seg_sum(e_pad, dh, s_pad):
    return _make_seg_sum(e_pad, dh, s_pad)


def _pad_edges(src, dst, e, e_pad, dummy_dst):
    srcp = jnp.concatenate(
        [src, jnp.zeros((e_pad - e,), jnp.int32)]).reshape(e_pad // _BATCH, _BATCH)
    dstp = jnp.concatenate(
        [dst, jnp.full((e_pad - e,), dummy_dst, jnp.int32)]).reshape(
            e_pad // _BATCH, _BATCH)
    return srcp, dstp


def kernel(features, src0, dst0, src1, dst1, W0, b0, W1, b1):
    src0p, dst0p = _pad_edges(src0, dst0, _E0, _E0_PAD, _N1)
    src1p, dst1p = _pad_edges(src1, dst1, _E1, _E1_PAD, _N2)
    zeros_h = jnp.zeros((_BATCH, _D_HID // 2), jnp.float32)
    zeros_o = jnp.zeros((_BATCH, _D_OUT // 2), jnp.float32)
    zeros_c = jnp.zeros((_BATCH, _CW), jnp.float32)
    ones_c = jnp.ones((_BATCH, _CW), jnp.float32)

    f = _matmul_split(features, W0)
    acc0, cnt0 = _seg_sum(_E0_PAD, _D_HID // 2, _S0_PAD)(
        f, src0p, dst0p, zeros_h, zeros_c, ones_c)
    g = _mid(acc0, cnt0, b0.reshape(1, _D_HID),
             W1[:_D_HID], W1[_D_HID:])
    acc1, cnt1 = _seg_sum(_E1_PAD, _D_OUT // 2, _S1_PAD)(
        g, src1p, dst1p, zeros_o, zeros_c, ones_c)
    return _fin(acc1, cnt1, b1.reshape(1, _D_OUT))
